# baseline (speedup 1.0000x reference)
import functools

import jax
import jax.numpy as jnp
from jax import lax
from jax.experimental import pallas as pl
from jax.experimental.pallas import tpu as pltpu
from jax.experimental.pallas import tpu_sc as plsc

D = 32
B = 16384
LANES = 16

NUM_CORES = 2
NUM_SUBCORES = 16
NUM_WORKERS = NUM_CORES * NUM_SUBCORES
COLS_PER_WORKER = B // NUM_WORKERS
TILE_W = 128
NBUF = 8

_mesh = plsc.VectorSubcoreMesh(core_axis_name="c", subcore_axis_name="s")


@functools.partial(
    pl.kernel,
    mesh=_mesh,
    out_type=jax.ShapeDtypeStruct((D, B), jnp.float32),
    scratch_types=[
        pltpu.VMEM((COLS_PER_WORKER,), jnp.int32),
        pltpu.VMEM((NBUF, D, TILE_W), jnp.float32),
        pltpu.VMEM((D, COLS_PER_WORKER), jnp.float32),
        pltpu.SemaphoreType.DMA((NBUF,)),
    ],
    compiler_params=pltpu.CompilerParams(needs_layout_passes=False),
)
def _sc_gather(table_hbm, idx_hbm, out_hbm, idx_v, blocks_v, cols_v, sems):
    wid = lax.axis_index("s") * NUM_CORES + lax.axis_index("c")
    base = wid * COLS_PER_WORKER
    pltpu.sync_copy(idx_hbm.at[pl.ds(base, COLS_PER_WORKER)], idx_v)

    def read_idx(b):
        vec = idx_v[pl.ds((b // LANES) * LANES, LANES)]
        lane = b % LANES
        mask = lax.iota(jnp.int32, LANES) == jnp.full((LANES,), lane, jnp.int32)
        return jnp.sum(jnp.where(mask, vec, 0))

    def issue(b, j):
        i = read_idx(b)
        c = pl.multiple_of((i // TILE_W) * TILE_W, TILE_W)
        pltpu.async_copy(
            table_hbm.at[:, pl.ds(c, TILE_W)], blocks_v.at[j], sems.at[j]
        )

    def drain(j):
        pltpu.make_async_copy(
            table_hbm.at[:, pl.ds(0, TILE_W)], blocks_v.at[j], sems.at[j]
        ).wait()

    def extract(b, j):
        i = read_idx(b)
        q = i % TILE_W
        rows = lax.iota(jnp.int32, LANES)
        colq = jnp.full((LANES,), q, jnp.int32)
        colb = jnp.full((LANES,), b, jnp.int32)
        for h in range(D // LANES):
            r = rows + h * LANES
            v = plsc.load_gather(blocks_v.at[j], [r, colq])
            plsc.store_scatter(cols_v, [r, colb], v)

    for j in range(NBUF):
        issue(j, j)

    def body(b0):
        for j in range(NBUF):
            drain(j)
        for j in range(NBUF):
            extract(b0 + j, j)
        for j in range(NBUF):
            @pl.when(b0 + j + NBUF < COLS_PER_WORKER)
            def _(j=j):
                issue(b0 + j + NBUF, j)

    pl.loop(0, COLS_PER_WORKER, step=NBUF)(body)
    pltpu.sync_copy(cols_v, out_hbm.at[:, pl.ds(base, COLS_PER_WORKER)])


def kernel(idx, z):
    out_t = _sc_gather(z.T, idx.astype(jnp.int32))
    return out_t.T


# speedup vs baseline: 1.4631x; 1.4631x over previous
import functools

import jax
import jax.numpy as jnp
from jax import lax
from jax.experimental import pallas as pl
from jax.experimental.pallas import tpu as pltpu
from jax.experimental.pallas import tpu_sc as plsc

D = 32
B = 16384
LANES = 16

NUM_CORES = 2
NUM_SUBCORES = 16
NUM_WORKERS = NUM_CORES * NUM_SUBCORES
COLS_PER_WORKER = B // NUM_WORKERS
TILE_W = 128
NBUF = 8

_mesh = plsc.VectorSubcoreMesh(core_axis_name="c", subcore_axis_name="s")


@functools.partial(
    pl.kernel,
    mesh=_mesh,
    out_type=jax.ShapeDtypeStruct((D, B), jnp.float32),
    scratch_types=[
        pltpu.VMEM((COLS_PER_WORKER,), jnp.int32),
        pltpu.VMEM((NBUF, D, TILE_W), jnp.float32),
        pltpu.VMEM((D, COLS_PER_WORKER), jnp.float32),
        pltpu.SemaphoreType.DMA((NBUF,)),
    ],
    compiler_params=pltpu.CompilerParams(needs_layout_passes=False),
)
def _sc_gather(table_hbm, idx_hbm, out_hbm, idx_v, blocks_v, cols_v, sems):
    wid = lax.axis_index("s") * NUM_CORES + lax.axis_index("c")
    base = wid * COLS_PER_WORKER
    pltpu.sync_copy(idx_hbm.at[pl.ds(base, COLS_PER_WORKER)], idx_v)

    def read_idx(b):
        s16 = plsc.load_gather(idx_v, [jnp.full((LANES,), b, jnp.int32)])
        return lax.squeeze(lax.slice(s16, (0,), (1,)), (0,))

    def issue(b, j):
        i = read_idx(b)
        c = pl.multiple_of((i // TILE_W) * TILE_W, TILE_W)
        pltpu.async_copy(
            table_hbm.at[:, pl.ds(c, TILE_W)], blocks_v.at[j], sems.at[j]
        )

    def drain_extract(b, j):
        pltpu.make_async_copy(
            table_hbm.at[:, pl.ds(0, TILE_W)], blocks_v.at[j], sems.at[j]
        ).wait()
        i = read_idx(b)
        q = i % TILE_W
        rows = lax.iota(jnp.int32, LANES)
        colq = jnp.full((LANES,), q, jnp.int32)
        colb = jnp.full((LANES,), b, jnp.int32)
        for h in range(D // LANES):
            r = rows + h * LANES
            v = plsc.load_gather(blocks_v.at[j], [r, colq])
            plsc.store_scatter(cols_v, [r, colb], v)

    for j in range(NBUF):
        issue(j, j)

    def body(b0):
        for j in range(NBUF):
            drain_extract(b0 + j, j)

            @pl.when(b0 + j + NBUF < COLS_PER_WORKER)
            def _():
                issue(b0 + j + NBUF, j)

    pl.loop(0, COLS_PER_WORKER, step=NBUF)(body)
    pltpu.sync_copy(cols_v, out_hbm.at[:, pl.ds(base, COLS_PER_WORKER)])


def kernel(idx, z):
    out_t = _sc_gather(z.T, idx.astype(jnp.int32))
    return out_t.T


# speedup vs baseline: 1.4851x; 1.0150x over previous
import functools

import jax
import jax.numpy as jnp
from jax import lax
from jax.experimental import pallas as pl
from jax.experimental.pallas import tpu as pltpu
from jax.experimental.pallas import tpu_sc as plsc

D = 32
B = 16384
LANES = 16

NUM_CORES = 2
NUM_SUBCORES = 16
NUM_WORKERS = NUM_CORES * NUM_SUBCORES
COLS_PER_WORKER = B // NUM_WORKERS
TILE_W = 128
NBUF = 8

_mesh = plsc.VectorSubcoreMesh(core_axis_name="c", subcore_axis_name="s")


@functools.partial(
    pl.kernel,
    mesh=_mesh,
    out_type=jax.ShapeDtypeStruct((D, B), jnp.float32),
    scratch_types=[
        pltpu.VMEM((COLS_PER_WORKER,), jnp.int32),
        pltpu.VMEM((NBUF, D, TILE_W), jnp.float32),
        pltpu.VMEM((D, COLS_PER_WORKER), jnp.float32),
        pltpu.SemaphoreType.DMA((NBUF,)),
    ],
    compiler_params=pltpu.CompilerParams(needs_layout_passes=False),
)
def _sc_gather(table_hbm, idx_hbm, out_hbm, idx_v, blocks_v, cols_v, sems):
    wid = lax.axis_index("s") * NUM_CORES + lax.axis_index("c")
    base = wid * COLS_PER_WORKER
    pltpu.sync_copy(idx_hbm.at[pl.ds(base, COLS_PER_WORKER)], idx_v)

    def read_idx(b):
        vec = idx_v[pl.ds((b // LANES) * LANES, LANES)]
        lane = b % LANES
        mask = lax.iota(jnp.int32, LANES) == jnp.full((LANES,), lane, jnp.int32)
        return jnp.sum(jnp.where(mask, vec, 0))

    def issue(b, j):
        i = read_idx(b)
        c = pl.multiple_of((i // TILE_W) * TILE_W, TILE_W)
        pltpu.async_copy(
            table_hbm.at[:, pl.ds(c, TILE_W)], blocks_v.at[j], sems.at[j]
        )

    def drain_extract(b, j):
        pltpu.make_async_copy(
            table_hbm.at[:, pl.ds(0, TILE_W)], blocks_v.at[j], sems.at[j]
        ).wait()
        i = read_idx(b)
        q = i % TILE_W
        rows = lax.iota(jnp.int32, LANES)
        colq = jnp.full((LANES,), q, jnp.int32)
        colb = jnp.full((LANES,), b, jnp.int32)
        for h in range(D // LANES):
            r = rows + h * LANES
            v = plsc.load_gather(blocks_v.at[j], [r, colq])
            plsc.store_scatter(cols_v, [r, colb], v)

    for j in range(NBUF):
        issue(j, j)

    def body(b0):
        for j in range(NBUF):
            drain_extract(b0 + j, j)

            @pl.when(b0 + j + NBUF < COLS_PER_WORKER)
            def _():
                issue(b0 + j + NBUF, j)

    pl.loop(0, COLS_PER_WORKER, step=NBUF)(body)
    pltpu.sync_copy(cols_v, out_hbm.at[:, pl.ds(base, COLS_PER_WORKER)])


def kernel(idx, z):
    out_t = _sc_gather(z.T, idx.astype(jnp.int32))
    return out_t.T
